# baseline (speedup 1.0000x reference)
import functools

import jax
import jax.numpy as jnp
from jax.experimental import pallas as pl

N = 116
NP = 128
F = 220
NHID = 20
NHEADS = 8
BS = 4


def _gat_kernel(x_ref, w1_ref, asr1_ref, adt1_ref, b1_ref,
                w2_ref, as2_ref, ad2_ref, b2_ref, out_ref):
    row_i = jax.lax.broadcasted_iota(jnp.int32, (NP, NP), 0)
    col_j = jax.lax.broadcasted_iota(jnp.int32, (NP, NP), 1)
    eye = row_i == col_j
    neg = jnp.float32(-1e9)

    w1 = w1_ref[...]
    asr1 = asr1_ref[...]
    adt1 = adt1_ref[...]
    w2 = w2_ref[...]

    for s in range(BS):
        x = x_ref[s]
        mean = jnp.sum(x, axis=1, keepdims=True) * (1.0 / F)
        xc = x - mean
        cov = jax.lax.dot_general(xc, xc, (((1,), (1,)), ((), ())))
        d = jnp.sqrt(jnp.sum(xc * xc, axis=1, keepdims=True))
        mask = (cov > 0.5 * d * jnp.transpose(d)) | eye

        h1 = jnp.dot(x, w1)
        a_srcT = jax.lax.dot_general(asr1, h1, (((0,), (1,)), ((), ())))
        a_dst = jnp.dot(h1, adt1)
        msgs = []
        for hd in range(NHEADS):
            lg = a_dst[:, hd:hd + 1] + a_srcT[hd:hd + 1, :]
            lg = jnp.where(lg >= 0, lg, 0.2 * lg)
            lg = jnp.where(mask, lg, neg)
            m = jnp.max(lg, axis=1, keepdims=True)
            e = jnp.exp(lg - m)
            alpha = e / jnp.sum(e, axis=1, keepdims=True)
            msgs.append(jnp.dot(alpha, h1[:, hd * NHID:(hd + 1) * NHID]))
        out1 = jnp.concatenate(msgs, axis=1) + b1_ref[...]
        out1 = jnp.where(out1 > 0, out1, jnp.exp(out1) - 1.0)

        h2 = jnp.dot(out1, w2)
        a2sT = jax.lax.dot_general(as2_ref[...], h2, (((1,), (1,)), ((), ())))
        a2d = jax.lax.dot_general(h2, ad2_ref[...], (((1,), (1,)), ((), ())))
        lg = a2d + a2sT
        lg = jnp.where(lg >= 0, lg, 0.2 * lg)
        lg = jnp.where(mask, lg, neg)
        m = jnp.max(lg, axis=1, keepdims=True)
        e = jnp.exp(lg - m)
        alpha = e / jnp.sum(e, axis=1, keepdims=True)
        out_ref[s, :, :] = jnp.dot(alpha, h2) + b2_ref[...]


def _mlp_kernel(flat_ref, l1w_ref, l1b_ref, g1_ref, bt1_ref,
                l2w_ref, l2b_ref, g2_ref, bt2_ref, l3w_ref, l3b_ref,
                probs_ref, block_ref):
    inv = jnp.float32(1.0 / (1.0 + 1e-5) ** 0.5)
    h = jnp.dot(flat_ref[...], l1w_ref[...]) + l1b_ref[...]
    h = h * (g1_ref[...] * inv) + bt1_ref[...]
    blk = jnp.dot(h, l2w_ref[...]) + l2b_ref[...]
    blk = blk * (g2_ref[...] * inv) + bt2_ref[...]
    block_ref[...] = blk
    lg = jnp.dot(blk, l3w_ref[...]) + l3b_ref[...]
    m = jnp.max(lg, axis=1, keepdims=True)
    e = jnp.exp(lg - m)
    probs_ref[...] = e / jnp.sum(e, axis=1, keepdims=True)


@jax.jit
def kernel(input, W1, as1, ad1, b1, W2, as2, ad2, b2, l1_w, l1_b,
           bn1_g, bn1_b, l2_w, l2_b, bn2_g, bn2_b, l3_w, l3_b):
    B = input.shape[0]
    xp = jnp.pad(input, ((0, 0), (0, NP - N), (0, 0)))
    eye8 = jnp.eye(NHEADS, dtype=jnp.float32)
    asr1 = (eye8[:, None, :] * as1[:, :, None]).reshape(NHEADS * NHID, NHEADS)
    adt1 = (eye8[:, None, :] * ad1[:, :, None]).reshape(NHEADS * NHID, NHEADS)

    full = lambda shp: pl.BlockSpec(shp, lambda i: (0,) * len(shp))
    gat = pl.pallas_call(
        _gat_kernel,
        grid=(B // BS,),
        in_specs=[
            pl.BlockSpec((BS, NP, F), lambda i: (i, 0, 0)),
            full((F, NHEADS * NHID)),
            full((NHEADS * NHID, NHEADS)),
            full((NHEADS * NHID, NHEADS)),
            full((1, NHEADS * NHID)),
            full((NHEADS * NHID, NHID)),
            full((1, NHID)),
            full((1, NHID)),
            full((1, NHID)),
        ],
        out_specs=pl.BlockSpec((BS, NP, NHID), lambda i: (i, 0, 0)),
        out_shape=jax.ShapeDtypeStruct((B, NP, NHID), jnp.float32),
    )(xp, W1, asr1, adt1, b1.reshape(1, -1), W2,
      as2.reshape(1, -1), ad2.reshape(1, -1), b2.reshape(1, -1))

    flat = gat[:, :N, :].reshape(B, N * NHID)
    probs, block = pl.pallas_call(
        _mlp_kernel,
        out_shape=(jax.ShapeDtypeStruct((B, 2), jnp.float32),
                   jax.ShapeDtypeStruct((B, 256), jnp.float32)),
    )(flat, l1_w, l1_b.reshape(1, -1), bn1_g.reshape(1, -1),
      bn1_b.reshape(1, -1), l2_w, l2_b.reshape(1, -1),
      bn2_g.reshape(1, -1), bn2_b.reshape(1, -1), l3_w, l3_b.reshape(1, -1))
    return probs, block


# speedup vs baseline: 1.3749x; 1.3749x over previous
import jax
import jax.numpy as jnp
from jax.experimental import pallas as pl

N = 116
NP = 128
F = 220
NHID = 20
NHEADS = 8
BS = 4


def _gat_kernel(x_ref, w1_ref, asr1_ref, adt1_ref, b1_ref,
                w2_ref, as2_ref, ad2_ref, b2_ref, out_ref):
    row_i = jax.lax.broadcasted_iota(jnp.int32, (NP, NP), 0)
    col_j = jax.lax.broadcasted_iota(jnp.int32, (NP, NP), 1)
    eyef = jnp.where(row_i == col_j, 1.0, 0.0).astype(jnp.float32)
    ones_col = jnp.ones((NP, 1), dtype=jnp.bfloat16)

    w1 = w1_ref[...]
    asr1 = asr1_ref[...]
    adt1 = adt1_ref[...]
    w2 = w2_ref[...]
    f32 = jnp.float32

    def leaky(v):
        return jnp.maximum(v, 0.2 * v)

    for s in range(BS):
        x = x_ref[s]
        mean = jnp.sum(x, axis=1, keepdims=True) * (1.0 / F)
        xc = x - mean
        cov = jax.lax.dot_general(xc, xc, (((1,), (1,)), ((), ())),
                                  preferred_element_type=f32)
        d = jnp.sqrt(jnp.sum(xc * xc, axis=1, keepdims=True))
        maskf = jnp.maximum(
            jnp.where(cov > 0.5 * d * jnp.transpose(d), 1.0, 0.0), eyef)

        h1 = jnp.dot(x, w1, preferred_element_type=f32)
        h1b = h1.astype(jnp.bfloat16)
        a_srcT = jax.lax.dot_general(asr1, h1, (((0,), (1,)), ((), ())),
                                     preferred_element_type=f32)
        a_dst = jnp.dot(h1, adt1, preferred_element_type=f32)
        msgs = []
        for hd in range(NHEADS):
            dst = a_dst[:, hd:hd + 1]
            src = a_srcT[hd:hd + 1, :]
            a_max = jnp.max(src, axis=1, keepdims=True)
            bound = leaky(a_max + dst)
            e = jnp.exp(leaky(dst + src) - bound) * maskf
            eb = e.astype(jnp.bfloat16)
            num = jax.lax.dot_general(eb, h1b[:, hd * NHID:(hd + 1) * NHID],
                                      (((1,), (0,)), ((), ())),
                                      preferred_element_type=f32)
            den = jax.lax.dot_general(eb, ones_col, (((1,), (0,)), ((), ())),
                                      preferred_element_type=f32)
            msgs.append(num * jax.lax.reciprocal(den))
        out1 = jnp.concatenate(msgs, axis=1) + b1_ref[...]
        out1 = jnp.where(out1 > 0, out1, jnp.exp(out1) - 1.0)

        h2 = jnp.dot(out1, w2, preferred_element_type=f32)
        a2sT = jax.lax.dot_general(as2_ref[...], h2, (((1,), (1,)), ((), ())),
                                   preferred_element_type=f32)
        a2d = jax.lax.dot_general(h2, ad2_ref[...], (((1,), (1,)), ((), ())),
                                  preferred_element_type=f32)
        bound2 = leaky(jnp.max(a2sT, axis=1, keepdims=True) + a2d)
        e2 = jnp.exp(leaky(a2d + a2sT) - bound2) * maskf
        e2b = e2.astype(jnp.bfloat16)
        num2 = jax.lax.dot_general(e2b, h2.astype(jnp.bfloat16),
                                   (((1,), (0,)), ((), ())),
                                   preferred_element_type=f32)
        den2 = jax.lax.dot_general(e2b, ones_col, (((1,), (0,)), ((), ())),
                                   preferred_element_type=f32)
        out_ref[s, :, :] = num2 * jax.lax.reciprocal(den2) + b2_ref[...]


def _mlp_kernel(flat_ref, l1w_ref, l1b_ref, g1_ref, bt1_ref,
                l2w_ref, l2b_ref, g2_ref, bt2_ref, l3w_ref, l3b_ref,
                probs_ref, block_ref):
    inv = jnp.float32(1.0 / (1.0 + 1e-5) ** 0.5)
    h = jnp.dot(flat_ref[...], l1w_ref[...]) + l1b_ref[...]
    h = h * (g1_ref[...] * inv) + bt1_ref[...]
    blk = jnp.dot(h, l2w_ref[...]) + l2b_ref[...]
    blk = blk * (g2_ref[...] * inv) + bt2_ref[...]
    block_ref[...] = blk
    lg = jnp.dot(blk, l3w_ref[...]) + l3b_ref[...]
    m = jnp.max(lg, axis=1, keepdims=True)
    e = jnp.exp(lg - m)
    probs_ref[...] = e / jnp.sum(e, axis=1, keepdims=True)


@jax.jit
def kernel(input, W1, as1, ad1, b1, W2, as2, ad2, b2, l1_w, l1_b,
           bn1_g, bn1_b, l2_w, l2_b, bn2_g, bn2_b, l3_w, l3_b):
    B = input.shape[0]
    xp = jnp.pad(input, ((0, 0), (0, NP - N), (0, 0)))
    eye8 = jnp.eye(NHEADS, dtype=jnp.float32)
    asr1 = (eye8[:, None, :] * as1[:, :, None]).reshape(NHEADS * NHID, NHEADS)
    adt1 = (eye8[:, None, :] * ad1[:, :, None]).reshape(NHEADS * NHID, NHEADS)

    full = lambda shp: pl.BlockSpec(shp, lambda i: (0,) * len(shp))
    gat = pl.pallas_call(
        _gat_kernel,
        grid=(B // BS,),
        in_specs=[
            pl.BlockSpec((BS, NP, F), lambda i: (i, 0, 0)),
            full((F, NHEADS * NHID)),
            full((NHEADS * NHID, NHEADS)),
            full((NHEADS * NHID, NHEADS)),
            full((1, NHEADS * NHID)),
            full((NHEADS * NHID, NHID)),
            full((1, NHID)),
            full((1, NHID)),
            full((1, NHID)),
        ],
        out_specs=pl.BlockSpec((BS, NP, NHID), lambda i: (i, 0, 0)),
        out_shape=jax.ShapeDtypeStruct((B, NP, NHID), jnp.float32),
    )(xp, W1, asr1, adt1, b1.reshape(1, -1), W2,
      as2.reshape(1, -1), ad2.reshape(1, -1), b2.reshape(1, -1))

    flat = gat[:, :N, :].reshape(B, N * NHID)
    probs, block = pl.pallas_call(
        _mlp_kernel,
        out_shape=(jax.ShapeDtypeStruct((B, 2), jnp.float32),
                   jax.ShapeDtypeStruct((B, 256), jnp.float32)),
    )(flat, l1_w, l1_b.reshape(1, -1), bn1_g.reshape(1, -1),
      bn1_b.reshape(1, -1), l2_w, l2_b.reshape(1, -1),
      bn2_g.reshape(1, -1), bn2_b.reshape(1, -1), l3_w, l3_b.reshape(1, -1))
    return probs, block


# speedup vs baseline: 1.4076x; 1.0238x over previous
import jax
import jax.numpy as jnp
from jax.experimental import pallas as pl

N = 116
NP = 128
F = 220
NHID = 20
NHEADS = 8
BS = 8


def _gat_kernel(x_ref, w1_ref, asr1_ref, adt1_ref, b1_ref,
                w2_ref, as2_ref, ad2_ref, b2_ref, out_ref):
    row_i = jax.lax.broadcasted_iota(jnp.int32, (NP, NP), 0)
    col_j = jax.lax.broadcasted_iota(jnp.int32, (NP, NP), 1)
    eyef = jnp.where(row_i == col_j, 1.0, 0.0).astype(jnp.float32)
    ones_col = jnp.ones((NP, 1), dtype=jnp.bfloat16)

    w1 = w1_ref[...]
    asr1 = asr1_ref[...]
    adt1 = adt1_ref[...]
    w2 = w2_ref[...]
    f32 = jnp.float32

    def leaky(v):
        return jnp.maximum(v, 0.2 * v)

    for s in range(BS):
        x = x_ref[s]
        mean = jnp.sum(x, axis=1, keepdims=True) * (1.0 / F)
        xc = x - mean
        cov = jax.lax.dot_general(xc, xc, (((1,), (1,)), ((), ())),
                                  preferred_element_type=f32)
        d = jnp.sqrt(jnp.sum(xc * xc, axis=1, keepdims=True))
        maskf = jnp.maximum(
            jnp.where(cov > 0.5 * d * jnp.transpose(d), 1.0, 0.0), eyef)

        h1 = jnp.dot(x, w1, preferred_element_type=f32)
        h1b = h1.astype(jnp.bfloat16)
        a_srcT = jax.lax.dot_general(asr1, h1, (((0,), (1,)), ((), ())),
                                     preferred_element_type=f32)
        a_dst = jnp.dot(h1, adt1, preferred_element_type=f32)
        msgs = []
        for hd in range(NHEADS):
            dst = a_dst[:, hd:hd + 1]
            src = a_srcT[hd:hd + 1, :]
            a_max = jnp.max(src, axis=1, keepdims=True)
            bound = leaky(a_max + dst)
            e = jnp.exp(leaky(dst + src) - bound) * maskf
            eb = e.astype(jnp.bfloat16)
            num = jax.lax.dot_general(eb, h1b[:, hd * NHID:(hd + 1) * NHID],
                                      (((1,), (0,)), ((), ())),
                                      preferred_element_type=f32)
            den = jax.lax.dot_general(eb, ones_col, (((1,), (0,)), ((), ())),
                                      preferred_element_type=f32)
            msgs.append(num * jax.lax.reciprocal(den))
        out1 = jnp.concatenate(msgs, axis=1) + b1_ref[...]
        out1 = jnp.where(out1 > 0, out1, jnp.exp(out1) - 1.0)

        h2 = jnp.dot(out1, w2, preferred_element_type=f32)
        a2sT = jax.lax.dot_general(as2_ref[...], h2, (((1,), (1,)), ((), ())),
                                   preferred_element_type=f32)
        a2d = jax.lax.dot_general(h2, ad2_ref[...], (((1,), (1,)), ((), ())),
                                  preferred_element_type=f32)
        bound2 = leaky(jnp.max(a2sT, axis=1, keepdims=True) + a2d)
        e2 = jnp.exp(leaky(a2d + a2sT) - bound2) * maskf
        e2b = e2.astype(jnp.bfloat16)
        num2 = jax.lax.dot_general(e2b, h2.astype(jnp.bfloat16),
                                   (((1,), (0,)), ((), ())),
                                   preferred_element_type=f32)
        den2 = jax.lax.dot_general(e2b, ones_col, (((1,), (0,)), ((), ())),
                                   preferred_element_type=f32)
        out_ref[s, :, :] = num2 * jax.lax.reciprocal(den2) + b2_ref[...]


def _mlp_kernel(flat_ref, l1w_ref, l1b_ref, g1_ref, bt1_ref,
                l2w_ref, l2b_ref, g2_ref, bt2_ref, l3w_ref, l3b_ref,
                probs_ref, block_ref):
    inv = jnp.float32(1.0 / (1.0 + 1e-5) ** 0.5)
    h = jnp.dot(flat_ref[...], l1w_ref[...]) + l1b_ref[...]
    h = h * (g1_ref[...] * inv) + bt1_ref[...]
    blk = jnp.dot(h, l2w_ref[...]) + l2b_ref[...]
    blk = blk * (g2_ref[...] * inv) + bt2_ref[...]
    block_ref[...] = blk
    lg = jnp.dot(blk, l3w_ref[...]) + l3b_ref[...]
    m = jnp.max(lg, axis=1, keepdims=True)
    e = jnp.exp(lg - m)
    probs_ref[...] = e / jnp.sum(e, axis=1, keepdims=True)


@jax.jit
def kernel(input, W1, as1, ad1, b1, W2, as2, ad2, b2, l1_w, l1_b,
           bn1_g, bn1_b, l2_w, l2_b, bn2_g, bn2_b, l3_w, l3_b):
    B = input.shape[0]
    xp = jnp.pad(input, ((0, 0), (0, NP - N), (0, 0)))
    eye8 = jnp.eye(NHEADS, dtype=jnp.float32)
    asr1 = (eye8[:, None, :] * as1[:, :, None]).reshape(NHEADS * NHID, NHEADS)
    adt1 = (eye8[:, None, :] * ad1[:, :, None]).reshape(NHEADS * NHID, NHEADS)

    full = lambda shp: pl.BlockSpec(shp, lambda i: (0,) * len(shp))
    gat = pl.pallas_call(
        _gat_kernel,
        grid=(B // BS,),
        in_specs=[
            pl.BlockSpec((BS, NP, F), lambda i: (i, 0, 0)),
            full((F, NHEADS * NHID)),
            full((NHEADS * NHID, NHEADS)),
            full((NHEADS * NHID, NHEADS)),
            full((1, NHEADS * NHID)),
            full((NHEADS * NHID, NHID)),
            full((1, NHID)),
            full((1, NHID)),
            full((1, NHID)),
        ],
        out_specs=pl.BlockSpec((BS, NP, NHID), lambda i: (i, 0, 0)),
        out_shape=jax.ShapeDtypeStruct((B, NP, NHID), jnp.float32),
    )(xp, W1, asr1, adt1, b1.reshape(1, -1), W2,
      as2.reshape(1, -1), ad2.reshape(1, -1), b2.reshape(1, -1))

    flat = gat[:, :N, :].reshape(B, N * NHID)
    probs, block = pl.pallas_call(
        _mlp_kernel,
        out_shape=(jax.ShapeDtypeStruct((B, 2), jnp.float32),
                   jax.ShapeDtypeStruct((B, 256), jnp.float32)),
    )(flat, l1_w, l1_b.reshape(1, -1), bn1_g.reshape(1, -1),
      bn1_b.reshape(1, -1), l2_w, l2_b.reshape(1, -1),
      bn2_g.reshape(1, -1), bn2_b.reshape(1, -1), l3_w, l3_b.reshape(1, -1))
    return probs, block
